# baseline (speedup 1.0000x reference)
import functools

import jax
import jax.numpy as jnp
from jax import lax
from jax.experimental import pallas as pl
from jax.experimental.pallas import tpu as pltpu
from jax.experimental.pallas import tpu_sc as plsc

B = 4
S = 2048
D = 768
N = B * S
NC = 2
NS = 16
NW = NC * NS
ROWS_PER_W = N // NW
CHUNK = 64
NCHUNK = ROWS_PER_W // CHUNK
LANES = 16
D_VECS = D // LANES


def _pe_table():
    pos = jnp.arange(S, dtype=jnp.float32)[:, None]
    i = jnp.arange(D // 2, dtype=jnp.float32)[None, :]
    angles = pos / jnp.power(10000.0, 2.0 * i / D)
    return jnp.stack([jnp.sin(angles), jnp.cos(angles)], axis=-1).reshape(S, D)


_mesh = plsc.VectorSubcoreMesh(core_axis_name="c", subcore_axis_name="s")


@functools.partial(
    pl.kernel,
    mesh=_mesh,
    out_type=jax.ShapeDtypeStruct((N, D), jnp.float32),
    scratch_types=[
        pltpu.VMEM((NCHUNK, CHUNK), jnp.int32),
        pltpu.VMEM((CHUNK, D), jnp.float32),
        pltpu.VMEM((CHUNK, D), jnp.float32),
        pltpu.SemaphoreType.DMA,
    ],
)
def _embed_pe(idx_hbm, table_hbm, pe_hbm, out_hbm, idx_v, rows_v, pe_v, sem):
    wid = lax.axis_index("s") * NC + lax.axis_index("c")
    base = wid * ROWS_PER_W
    pe_base = lax.rem(base, S)

    pltpu.sync_copy(idx_hbm.at[wid], idx_v)

    for c in range(NCHUNK):
        gather = pltpu.async_copy(table_hbm.at[idx_v.at[c]], rows_v, sem)
        pltpu.sync_copy(pe_hbm.at[pl.ds(pe_base + c * CHUNK, CHUNK)], pe_v)
        gather.wait()

        def row_add(i, carry):
            for j in range(D_VECS):
                sl = pl.ds(j * LANES, LANES)
                rows_v[i, sl] = rows_v[i, sl] + pe_v[i, sl]
            return carry

        lax.fori_loop(0, CHUNK, row_add, 0)

        pltpu.sync_copy(rows_v, out_hbm.at[pl.ds(base + c * CHUNK, CHUNK)])


def kernel(input, table):
    ids = input.reshape(N).astype(jnp.int32).reshape(NW, NCHUNK, CHUNK)
    pe = _pe_table()
    out = _embed_pe(ids, table, pe)
    return out.reshape(B, S, D)


# speedup vs baseline: 1.2151x; 1.2151x over previous
import functools

import jax
import jax.numpy as jnp
from jax import lax
from jax.experimental import pallas as pl
from jax.experimental.pallas import tpu as pltpu
from jax.experimental.pallas import tpu_sc as plsc

B = 4
S = 2048
D = 768
N = B * S
NC = 2
NS = 16
NW = NC * NS
POS_PER_W = S // NW
H = 2
CH = POS_PER_W // H
NCHUNK = H * B
LANES = 16
D_VECS = D // LANES


def _pe_table():
    pos = jnp.arange(S, dtype=jnp.float32)[:, None]
    i = jnp.arange(D // 2, dtype=jnp.float32)[None, :]
    angles = pos / jnp.power(10000.0, 2.0 * i / D)
    return jnp.stack([jnp.sin(angles), jnp.cos(angles)], axis=-1).reshape(S, D)


_mesh = plsc.VectorSubcoreMesh(core_axis_name="c", subcore_axis_name="s")


@functools.partial(
    pl.kernel,
    mesh=_mesh,
    out_type=jax.ShapeDtypeStruct((N, D), jnp.float32),
    scratch_types=[
        pltpu.VMEM((H, B, CH), jnp.int32),
        pltpu.VMEM((CH, D), jnp.float32),
        pltpu.VMEM((2, CH, D), jnp.float32),
        pltpu.SemaphoreType.DMA,
        pltpu.SemaphoreType.DMA,
        pltpu.SemaphoreType.DMA,
        pltpu.SemaphoreType.DMA,
    ],
)
def _embed_pe(idx_hbm, table_hbm, pe_hbm, out_hbm,
              idx_v, pe_v, rows_v, g_sem0, g_sem1, s_sem0, s_sem1):
    wid = lax.axis_index("s") * NC + lax.axis_index("c")
    base = wid * POS_PER_W

    pltpu.sync_copy(idx_hbm.at[wid], idx_v)

    g_sems = (g_sem0, g_sem1)
    s_sems = (s_sem0, s_sem1)
    chunks = [(h, b) for h in range(H) for b in range(B)]
    gathers = [None] * NCHUNK
    stores = [None] * NCHUNK

    def issue_gather(k):
        h, b = chunks[k]
        p = k % 2
        gathers[k] = pltpu.async_copy(
            table_hbm.at[idx_v.at[h, b]], rows_v.at[p], g_sems[p])

    issue_gather(0)
    pe_loaded = -1
    for k in range(NCHUNK):
        h, b = chunks[k]
        p = k % 2
        if h != pe_loaded:
            pltpu.sync_copy(pe_hbm.at[pl.ds(base + h * CH, CH)], pe_v)
            pe_loaded = h
        gathers[k].wait()
        if k + 1 < NCHUNK:
            if k >= 1 and stores[k - 1] is not None:
                stores[k - 1].wait()
            issue_gather(k + 1)

        def row_add(i, carry):
            for j in range(D_VECS):
                sl = pl.ds(j * LANES, LANES)
                rows_v[p, i, sl] = rows_v[p, i, sl] + pe_v[i, sl]
            return carry

        lax.fori_loop(0, CH, row_add, 0)
        stores[k] = pltpu.async_copy(
            rows_v.at[p], out_hbm.at[pl.ds(b * S + base + h * CH, CH)],
            s_sems[p])
    stores[NCHUNK - 2].wait()
    stores[NCHUNK - 1].wait()


def kernel(input, table):
    ids = (input.astype(jnp.int32)
           .reshape(B, NW, H, CH)
           .transpose(1, 2, 0, 3))
    pe = _pe_table()
    out = _embed_pe(ids, table, pe)
    return out.reshape(B, S, D)
